# Initial kernel scaffold; baseline (speedup 1.0000x reference)
import jax
import jax.numpy as jnp
from jax.experimental import pallas as pl


def kernel(img_rep_tokens, table):
    raise NotImplementedError("write your pallas kernel here")


# speedup vs baseline: 3.4095x; 3.4095x over previous
import functools

import jax
import jax.numpy as jnp
from jax import lax
from jax.experimental import pallas as pl
from jax.experimental.pallas import tpu as pltpu
from jax.experimental.pallas import tpu_sc as plsc

DIM = 64
NUM_CORES = 2
NUM_SUBCORES = 16
NW = NUM_CORES * NUM_SUBCORES


def _make_sc_gather(n_total: int):
    per_w = n_total // NW
    chunk = 128
    n_chunks = per_w // chunk

    mesh = plsc.VectorSubcoreMesh(core_axis_name="c", subcore_axis_name="s")

    @functools.partial(
        pl.kernel,
        mesh=mesh,
        compiler_params=pltpu.CompilerParams(use_tc_tiling_on_sc=False),
        out_type=jax.ShapeDtypeStruct((n_total, DIM), jnp.float32),
        scratch_types=[
            pltpu.VMEM((n_chunks, chunk), jnp.int32),
            pltpu.VMEM((chunk, DIM), jnp.float32),
            pltpu.SemaphoreType.DMA,
        ],
    )
    def k(idx_hbm, table_hbm, out_hbm, idx_v, rows_v, gsem):
        wid = lax.axis_index("s") * NUM_CORES + lax.axis_index("c")
        base = wid * per_w
        pltpu.sync_copy(idx_hbm.at[wid], idx_v)

        def body(j, carry):
            pltpu.async_copy(table_hbm.at[idx_v.at[j]], rows_v, gsem).wait()
            pltpu.sync_copy(rows_v, out_hbm.at[pl.ds(base + j * chunk, chunk)])
            return carry

        lax.fori_loop(0, n_chunks, body, 0)

    return k


def kernel(img_rep_tokens, table):
    b, f = img_rep_tokens.shape
    n = b * f
    idx3d = img_rep_tokens.reshape(NW, n // (NW * 128), 128)
    out = _make_sc_gather(n)(idx3d, table)
    return out.reshape(b, f, DIM)


# speedup vs baseline: 3.4984x; 1.0261x over previous
import functools

import jax
import jax.numpy as jnp
from jax import lax
from jax.experimental import pallas as pl
from jax.experimental.pallas import tpu as pltpu
from jax.experimental.pallas import tpu_sc as plsc

DIM = 64
NUM_CORES = 2
NUM_SUBCORES = 16
NW = NUM_CORES * NUM_SUBCORES
CHUNK = 128
NBUF = 8


def _make_sc_gather(n_total: int):
    per_w = n_total // NW
    n_chunks = per_w // CHUNK
    n_groups = n_chunks // NBUF

    mesh = plsc.VectorSubcoreMesh(core_axis_name="c", subcore_axis_name="s")

    @functools.partial(
        pl.kernel,
        mesh=mesh,
        compiler_params=pltpu.CompilerParams(use_tc_tiling_on_sc=False),
        out_type=jax.ShapeDtypeStruct((n_total, DIM), jnp.float32),
        scratch_types=[
            pltpu.VMEM((n_chunks, CHUNK), jnp.int32),
            pltpu.VMEM((NBUF, CHUNK, DIM), jnp.float32),
            pltpu.SemaphoreType.DMA((NBUF,)),
            pltpu.SemaphoreType.DMA((NBUF,)),
        ],
    )
    def k(idx_hbm, table_hbm, out_hbm, idx_v, bufs, gsems, wsems):
        wid = lax.axis_index("s") * NUM_CORES + lax.axis_index("c")
        base = wid * per_w
        pltpu.sync_copy(idx_hbm.at[wid], idx_v)

        def gather(j, b):
            pltpu.async_copy(table_hbm.at[idx_v.at[j]], bufs.at[b], gsems.at[b])

        def gather_wait(b):
            pltpu.make_async_copy(
                table_hbm.at[idx_v.at[0]], bufs.at[b], gsems.at[b]
            ).wait()

        def write(j, b):
            pltpu.async_copy(
                bufs.at[b], out_hbm.at[pl.ds(base + j * CHUNK, CHUNK)], wsems.at[b]
            )

        def write_wait(b):
            pltpu.make_async_copy(
                out_hbm.at[pl.ds(base, CHUNK)], bufs.at[b], wsems.at[b]
            ).wait()

        for b in range(NBUF):
            gather(b, b)

        def body(i, carry):
            jg = i * NBUF
            for b in range(NBUF):
                gather_wait(b)
                write(jg + b, b)
            for b in range(NBUF):
                write_wait(b)
                gather(jg + NBUF + b, b)
            return carry

        lax.fori_loop(0, n_groups - 1, body, 0)

        jg = (n_groups - 1) * NBUF
        for b in range(NBUF):
            gather_wait(b)
            write(jg + b, b)
        for b in range(NBUF):
            write_wait(b)

    return k


def kernel(img_rep_tokens, table):
    b, f = img_rep_tokens.shape
    n = b * f
    idx3d = img_rep_tokens.reshape(NW, n // (NW * CHUNK), CHUNK)
    out = _make_sc_gather(n)(idx3d, table)
    return out.reshape(b, f, DIM)


# speedup vs baseline: 8.9194x; 2.5496x over previous
import functools

import jax
import jax.numpy as jnp
from jax import lax
from jax.experimental import pallas as pl
from jax.experimental.pallas import tpu as pltpu
from jax.experimental.pallas import tpu_sc as plsc

VOCAB = 1000
DIM = 64
NUM_CORES = 2
NUM_SUBCORES = 16
NW = NUM_CORES * NUM_SUBCORES
CHUNK = 128
NBUF = 4


def _make_sc_gather(n_total: int):
    per_w = n_total // NW
    n_chunks = per_w // CHUNK
    n_groups = n_chunks // NBUF

    mesh = plsc.VectorSubcoreMesh(core_axis_name="c", subcore_axis_name="s")

    @functools.partial(
        pl.kernel,
        mesh=mesh,
        out_type=jax.ShapeDtypeStruct((n_total, DIM), jnp.float32),
        scratch_types=[
            pltpu.VMEM((n_chunks, CHUNK), jnp.int32),
            pltpu.VMEM((NBUF, CHUNK, DIM), jnp.float32),
            pltpu.VMEM_SHARED((VOCAB, DIM), jnp.float32),
            pltpu.SemaphoreType.DMA((NBUF,)),
            pltpu.SemaphoreType.DMA((NBUF,)),
        ],
    )
    def k(idx_hbm, table_hbm, out_hbm, idx_v, bufs, table_sh, gsems, wsems):
        cid = lax.axis_index("c")
        sid = lax.axis_index("s")
        wid = sid * NUM_CORES + cid
        base = wid * per_w

        @pl.when(sid == 0)
        def _():
            pltpu.sync_copy(table_hbm, table_sh)

        pltpu.sync_copy(idx_hbm.at[wid], idx_v)
        plsc.subcore_barrier()

        def gather(j, b):
            pltpu.async_copy(table_sh.at[idx_v.at[j]], bufs.at[b], gsems.at[b])

        def gather_wait(b):
            pltpu.make_async_copy(
                table_sh.at[idx_v.at[0]], bufs.at[b], gsems.at[b]
            ).wait()

        def write(j, b):
            pltpu.async_copy(
                bufs.at[b], out_hbm.at[pl.ds(base + j * CHUNK, CHUNK)], wsems.at[b]
            )

        def write_wait(b):
            pltpu.make_async_copy(
                out_hbm.at[pl.ds(base, CHUNK)], bufs.at[b], wsems.at[b]
            ).wait()

        for b in range(NBUF):
            gather(b, b)

        def body(i, carry):
            jg = i * NBUF
            for b in range(NBUF):
                gather_wait(b)
                write(jg + b, b)
            for b in range(NBUF):
                write_wait(b)
                gather(jg + NBUF + b, b)
            return carry

        lax.fori_loop(0, n_groups - 1, body, 0)

        jg = (n_groups - 1) * NBUF
        for b in range(NBUF):
            gather_wait(b)
            write(jg + b, b)
        for b in range(NBUF):
            write_wait(b)

    return k


def kernel(img_rep_tokens, table):
    b, f = img_rep_tokens.shape
    n = b * f
    idx3d = img_rep_tokens.reshape(NW, n // (NW * CHUNK), CHUNK)
    out = _make_sc_gather(n)(idx3d, table)
    return out.reshape(b, f, DIM)
